# baseline (speedup 1.0000x reference)
import functools

import jax
import jax.numpy as jnp
from jax import lax
from jax.experimental import pallas as pl
from jax.experimental.pallas import tpu as pltpu
from jax.experimental.pallas import tpu_sc as plsc

N = 10000
C = 128
H = 8
NEG_SLOPE = 0.2

NPAD = 10240
ROWS_PER_TILE = NPAD // 16
K = 128
NW = 32

_MESH = plsc.VectorSubcoreMesh(core_axis_name="c", subcore_axis_name="s")


def _backbone_body(nchunks, xs_h, src_h, dst_h, z2d_h, z16_h, eye_h,
                   outagg_h, outdeg_h,
                   idxS0, idxD0, bufX0, idxS1, idxD1, bufX1, ones_v, sem,
                   agg_sh, deg_sh):
    c = lax.axis_index("c")
    s = lax.axis_index("s")
    row0 = s * ROWS_PER_TILE
    for i in range(8):
        pltpu.sync_copy(z2d_h, agg_sh.at[pl.ds(row0 + i * (ROWS_PER_TILE // 8), ROWS_PER_TILE // 8), :])
    pltpu.sync_copy(z16_h, deg_sh.at[pl.ds(row0, ROWS_PER_TILE), :])
    pltpu.sync_copy(eye_h.at[c], ones_v)
    plsc.subcore_barrier()

    base_w = s * (nchunks * K)

    def issue(g, idxS, idxD, bufX):
        base = base_w + g * K
        pltpu.sync_copy(src_h.at[pl.ds(base, K)], idxS)
        pltpu.sync_copy(dst_h.at[pl.ds(base, K)], idxD)
        pltpu.async_copy(xs_h.at[c].at[idxS], bufX, sem)

    def process(idxS, idxD, bufX):
        pltpu.make_async_copy(xs_h.at[c].at[idxS], bufX, sem).wait()
        pltpu.sync_copy(bufX, agg_sh.at[idxD], add=True)

        @pl.when(c == 0)
        def _():
            pltpu.sync_copy(ones_v, deg_sh.at[idxD], add=True)

        @pl.when(c == 1)
        def _():
            pltpu.sync_copy(ones_v, deg_sh.at[idxS], add=True)

    issue(0, idxS0, idxD0, bufX0)

    def pair(h, carry):
        g0 = 2 * h
        issue(g0 + 1, idxS1, idxD1, bufX1)
        process(idxS0, idxD0, bufX0)

        @pl.when(g0 + 2 < nchunks)
        def _():
            issue(g0 + 2, idxS0, idxD0, bufX0)

        process(idxS1, idxD1, bufX1)
        return carry

    lax.fori_loop(0, nchunks // 2, pair, 0)
    plsc.subcore_barrier()
    pltpu.sync_copy(agg_sh.at[pl.ds(row0, ROWS_PER_TILE), :],
                    outagg_h.at[c, pl.ds(row0, ROWS_PER_TILE), :])
    pltpu.sync_copy(deg_sh.at[pl.ds(row0, ROWS_PER_TILE), :],
                    outdeg_h.at[c, pl.ds(row0, ROWS_PER_TILE), :])


def _backbone_sc(xsplit, srcp, dstp, nchunks):
    kfn = pl.kernel(
        functools.partial(_backbone_body, nchunks),
        out_type=[
            jax.ShapeDtypeStruct((2, NPAD, C // 2), jnp.float32),
            jax.ShapeDtypeStruct((2, NPAD, 16), jnp.float32),
        ],
        mesh=_MESH,
        compiler_params=pltpu.CompilerParams(use_tc_tiling_on_sc=False),
        scratch_types=[
            pltpu.VMEM((K,), jnp.int32),
            pltpu.VMEM((K,), jnp.int32),
            pltpu.VMEM((K, C // 2), jnp.float32),
            pltpu.VMEM((K,), jnp.int32),
            pltpu.VMEM((K,), jnp.int32),
            pltpu.VMEM((K, C // 2), jnp.float32),
            pltpu.VMEM((K, 16), jnp.float32),
            pltpu.SemaphoreType.DMA,
            pltpu.VMEM_SHARED((NPAD, C // 2), jnp.float32),
            pltpu.VMEM_SHARED((NPAD, 16), jnp.float32),
        ],
    )
    z2d = jnp.zeros((ROWS_PER_TILE // 8, C // 2), jnp.float32)
    z16 = jnp.zeros((ROWS_PER_TILE, 16), jnp.float32)
    eye = jnp.zeros((2, K, 16), jnp.float32).at[0, :, 0].set(1.0).at[1, :, 1].set(1.0)
    return kfn(xsplit, srcp, dstp, z2d, z16, eye)


RG = NPAD // 16
RT = RG // 16

_GDN = lax.GatherDimensionNumbers(offset_dims=(), collapsed_slice_dims=(0,),
                                  start_index_map=(0,))


def _take16(v, perm):
    return lax.gather(v, perm[:, None], _GDN, slice_sizes=(1,),
                      mode=lax.GatherScatterMode.PROMISE_IN_BOUNDS)


def _attn_body(nchunks, af_h, src_h, dst_h, ae_h, outmax_h,
               idxS0, idxD0, bufS0, bufD0, idxS1, idxD1, bufS1, bufD1,
               aebuf, sem, locmax, red, tmpv, spmax_sh):
    c = lax.axis_index("c")
    s = lax.axis_index("s")
    w = s * 2 + c

    neg = jnp.full((16,), -3.0e38, jnp.float32)

    def initrow(i, car):
        locmax[pl.ds(i * 16, 16)] = neg
        return car

    lax.fori_loop(0, RG, initrow, 0)

    base_w = w * nchunks * K
    lane = lax.iota(jnp.int32, 16)

    def issue(g, idxS, idxD, bufS, bufD):
        base = base_w + g * K
        pltpu.sync_copy(src_h.at[pl.ds(base, K)], idxS)
        pltpu.sync_copy(dst_h.at[pl.ds(base, K)], idxD)
        pltpu.async_copy(af_h.at[idxS], bufS, sem)
        pltpu.async_copy(af_h.at[idxD], bufD, sem)

    def compute(g, idxS, idxD, bufS, bufD):
        base = base_w + g * K
        pltpu.make_async_copy(af_h.at[idxS], bufS, sem).wait()
        pltpu.make_async_copy(af_h.at[idxD], bufD, sem).wait()
        bfly = [lax.bitwise_xor(lane, sh) for sh in (1, 2, 4, 8)]
        rots = [lax.bitwise_and(lane + r, 15) for r in range(1, 16)]

        def group(g2, car2):
            e0 = g2 * 16
            vals = jnp.zeros((16,), jnp.float32)
            for j in range(16):
                i = e0 + j
                p = bufS[i, pl.ds(0, 16)] * bufD[i, pl.ds(0, 16)]
                for b in range(1, 8):
                    p = p + bufS[i, pl.ds(16 * b, 16)] * bufD[i, pl.ds(16 * b, 16)]
                for pm in bfly:
                    p = p + _take16(p, pm)
                vals = jnp.where(lane == j, p, vals)
            vals = jnp.maximum(vals, NEG_SLOPE * vals)
            aebuf[pl.ds(e0, 16)] = vals
            dvec = idxD[pl.ds(e0, 16)]
            mv = vals
            for pm in rots:
                mv = jnp.where(_take16(dvec, pm) == dvec,
                               jnp.maximum(mv, _take16(mv, pm)), mv)
            cur = plsc.load_gather(locmax, [dvec])
            plsc.store_scatter(locmax, [dvec], jnp.maximum(cur, mv))
            return car2

        lax.fori_loop(0, K // 16, group, 0)
        pltpu.sync_copy(aebuf, ae_h.at[pl.ds(base, K)])

    issue(0, idxS0, idxD0, bufS0, bufD0)

    def pair(h, car):
        g0 = 2 * h
        issue(g0 + 1, idxS1, idxD1, bufS1, bufD1)
        compute(g0, idxS0, idxD0, bufS0, bufD0)

        @pl.when(g0 + 2 < nchunks)
        def _():
            issue(g0 + 2, idxS0, idxD0, bufS0, bufD0)

        compute(g0 + 1, idxS1, idxD1, bufS1, bufD1)
        return car

    lax.fori_loop(0, nchunks // 2, pair, 0)
    pltpu.sync_copy(locmax, spmax_sh.at[s])
    plsc.subcore_barrier()
    r0 = s * RT
    pltpu.sync_copy(spmax_sh.at[0, pl.ds(r0 * 16, RT * 16)], red)
    for t in range(1, 16):
        pltpu.sync_copy(spmax_sh.at[t, pl.ds(r0 * 16, RT * 16)], tmpv)

        def mrow(i, car):
            red[pl.ds(i * 16, 16)] = jnp.maximum(red[pl.ds(i * 16, 16)], tmpv[pl.ds(i * 16, 16)])
            return car

        lax.fori_loop(0, RT, mrow, 0)
    pltpu.sync_copy(red, outmax_h.at[c, pl.ds(r0 * 16, RT * 16)])


def _attn_sc(af, srcp, dstp, nchunks, epp):
    kfn = pl.kernel(
        functools.partial(_attn_body, nchunks),
        out_type=[
            jax.ShapeDtypeStruct((epp,), jnp.float32),
            jax.ShapeDtypeStruct((2, NPAD), jnp.float32),
        ],
        mesh=_MESH,
        compiler_params=pltpu.CompilerParams(use_tc_tiling_on_sc=False,
                                             needs_layout_passes=False),
        scratch_types=[
            pltpu.VMEM((K,), jnp.int32),
            pltpu.VMEM((K,), jnp.int32),
            pltpu.VMEM((K, C), jnp.float32),
            pltpu.VMEM((K, C), jnp.float32),
            pltpu.VMEM((K,), jnp.int32),
            pltpu.VMEM((K,), jnp.int32),
            pltpu.VMEM((K, C), jnp.float32),
            pltpu.VMEM((K, C), jnp.float32),
            pltpu.VMEM((K,), jnp.float32),
            pltpu.SemaphoreType.DMA,
            pltpu.VMEM((NPAD,), jnp.float32),
            pltpu.VMEM((RT * 16,), jnp.float32),
            pltpu.VMEM((RT * 16,), jnp.float32),
            pltpu.VMEM_SHARED((16, NPAD), jnp.float32),
        ],
    )
    return kfn(af, srcp, dstp)


def _soft_body(nchunks, packed_h, src_h, dst_h, ae_h, maxp_h, z16_h, msk_h,
               outacc_h,
               idxS, idxD, bufP, aev, sem, aml, tmpv, mskv, acc_sh):
    c = lax.axis_index("c")
    s = lax.axis_index("s")
    w = s * 2 + c
    pltpu.sync_copy(maxp_h.at[0], aml)
    pltpu.sync_copy(maxp_h.at[1], tmpv)

    def mrow(i, car):
        aml[pl.ds(i * 16, 16)] = jnp.maximum(aml[pl.ds(i * 16, 16)], tmpv[pl.ds(i * 16, 16)])
        return car

    lax.fori_loop(0, RG, mrow, 0)
    row0 = s * ROWS_PER_TILE
    pltpu.sync_copy(z16_h, acc_sh.at[pl.ds(row0, ROWS_PER_TILE), :])
    pltpu.sync_copy(msk_h, mskv)
    plsc.subcore_barrier()
    mA = mskv[0]
    mB = mskv[1]
    base_w = w * nchunks * K

    def chunk(g, car):
        base = base_w + g * K
        pltpu.sync_copy(src_h.at[pl.ds(base, K)], idxS)
        pltpu.sync_copy(dst_h.at[pl.ds(base, K)], idxD)
        pltpu.async_copy(packed_h.at[idxS], bufP, sem).wait()
        pltpu.sync_copy(ae_h.at[pl.ds(base, K)], aev)

        def group(g2, car2):
            e0 = g2 * 16
            dvec = idxD[pl.ds(e0, 16)]
            am = plsc.load_gather(aml, [dvec])
            ex = jnp.exp(aev[pl.ds(e0, 16)] - am)
            for j in range(16):
                i = e0 + j
                t = ex[j] * mA + mB
                bufP[i, :] = bufP[i, :] * t
            return car2

        lax.fori_loop(0, K // 16, group, 0)
        pltpu.sync_copy(bufP, acc_sh.at[idxD], add=True)
        return car

    lax.fori_loop(0, nchunks, chunk, 0)
    plsc.subcore_barrier()
    pltpu.sync_copy(acc_sh.at[pl.ds(row0, ROWS_PER_TILE), :],
                    outacc_h.at[c, pl.ds(row0, ROWS_PER_TILE), :])


def _soft_sc(packed, srcp, dstp, ae, maxp, nchunks):
    kfn = pl.kernel(
        functools.partial(_soft_body, nchunks),
        out_type=jax.ShapeDtypeStruct((2, NPAD, 16), jnp.float32),
        mesh=_MESH,
        compiler_params=pltpu.CompilerParams(use_tc_tiling_on_sc=False,
                                             needs_layout_passes=False),
        scratch_types=[
            pltpu.VMEM((K,), jnp.int32),
            pltpu.VMEM((K,), jnp.int32),
            pltpu.VMEM((K, 16), jnp.float32),
            pltpu.VMEM((K,), jnp.float32),
            pltpu.SemaphoreType.DMA,
            pltpu.VMEM((NPAD,), jnp.float32),
            pltpu.VMEM((NPAD,), jnp.float32),
            pltpu.VMEM((2, 16), jnp.float32),
            pltpu.VMEM_SHARED((NPAD, 16), jnp.float32),
        ],
    )
    z16 = jnp.zeros((ROWS_PER_TILE, 16), jnp.float32)
    msk = jnp.zeros((2, 16), jnp.float32).at[0, 0:8].set(1.0).at[0, 9].set(1.0).at[1, 8].set(1.0)
    return kfn(packed, srcp, dstp, ae, maxp, z16, msk)


BR = 512


def _sort_lanes(v):
    li = lax.broadcasted_iota(jnp.int32, v.shape, 1)
    k = 2
    while k <= 128:
        j = k // 2
        while j >= 1:
            bitj0 = (li & j) == 0
            p = jnp.where(bitj0, jnp.roll(v, -j, axis=1), jnp.roll(v, j, axis=1))
            dirup = (li & k) == 0
            keepmin = bitj0 == dirup
            v = jnp.where(keepmin, jnp.minimum(v, p), jnp.maximum(v, p))
            j //= 2
        k *= 2
    return v


def _node_body(aggp_ref, degp_ref, w_ref, wt_ref, b_ref, dist_ref, ta_ref, da_ref,
               logits_ref, af_ref, packed_ref):
    agg = jnp.concatenate([aggp_ref[0], aggp_ref[1]], axis=1)
    din = degp_ref[0, :, 0:1]
    dout = degp_ref[1, :, 1:2]
    logits = (agg / jnp.maximum(din, 1.0)) @ w_ref[...] + b_ref[...]
    logits_ref[...] = logits
    mn = jnp.min(logits, axis=1, keepdims=True)
    mx = jnp.max(logits, axis=1, keepdims=True)
    normalized = (logits - mn) / jnp.maximum(mx - mn, 1e-30)
    x_sorted = _sort_lanes(normalized)
    temp = jax.lax.dot(x_sorted, wt_ref[...], preferred_element_type=jnp.float32)
    m = jnp.max(logits, axis=1, keepdims=True)
    conf = 1.0 / jnp.sum(jnp.exp(logits - m), axis=1, keepdims=True)
    dist = dist_ref[...]
    a = jnp.where(dist == 0, ta_ref[0], jnp.where(dist == 1, da_ref[0], 1.0))
    af_ref[...] = logits * (1.0 / a)
    dinv = jnp.where(dout > 0, 1.0 / dout, 0.0)
    zs = jnp.zeros_like(temp[:, 0:4])
    packed_ref[...] = jnp.concatenate(
        [temp[:, 0:8] * a, conf, jnp.ones_like(conf), din, dinv, zs], axis=1)


def _node_tc(aggp, degp, W_model, W_temp, b_model, dist_pad, train_a, dist1_a):
    wt_pad = jnp.zeros((C, C), jnp.float32).at[:, :H].set(W_temp)
    return pl.pallas_call(
        _node_body,
        grid=(NPAD // BR,),
        in_specs=[
            pl.BlockSpec((2, BR, C // 2), lambda i: (0, i, 0)),
            pl.BlockSpec((2, BR, 16), lambda i: (0, i, 0)),
            pl.BlockSpec((C, C), lambda i: (0, 0)),
            pl.BlockSpec((C, C), lambda i: (0, 0)),
            pl.BlockSpec((1, C), lambda i: (0, 0)),
            pl.BlockSpec((BR, 1), lambda i: (i, 0)),
            pl.BlockSpec(memory_space=pltpu.SMEM),
            pl.BlockSpec(memory_space=pltpu.SMEM),
        ],
        out_specs=[
            pl.BlockSpec((BR, C), lambda i: (i, 0)),
            pl.BlockSpec((BR, C), lambda i: (i, 0)),
            pl.BlockSpec((BR, 16), lambda i: (i, 0)),
        ],
        out_shape=[
            jax.ShapeDtypeStruct((NPAD, C), jnp.float32),
            jax.ShapeDtypeStruct((NPAD, C), jnp.float32),
            jax.ShapeDtypeStruct((NPAD, 16), jnp.float32),
        ],
    )(aggp, degp, W_model, wt_pad, b_model.reshape(1, C), dist_pad,
      train_a, dist1_a)


def _final_body(accp_ref, packed_ref, logits_ref, coef_ref, bias_ref, out_ref):
    accs = accp_ref[0] + accp_ref[1]
    sim = accs[:, 0:8] / accs[:, 9:10]
    conf = packed_ref[:, 8:9]
    din = packed_ref[:, 10:11]
    dinv = packed_ref[:, 11:12]
    dconf = din * conf - accs[:, 8:9]
    out = jax.nn.softplus(sim + coef_ref[0] * dconf * dinv)
    t = jnp.sum(out, axis=1, keepdims=True) * (1.0 / H) + bias_ref[0]
    out_ref[...] = logits_ref[...] / t


def _final_tc(accp, packed, logits, conf_coef, bias_p):
    return pl.pallas_call(
        _final_body,
        grid=(NPAD // BR,),
        in_specs=[
            pl.BlockSpec((2, BR, 16), lambda i: (0, i, 0)),
            pl.BlockSpec((BR, 16), lambda i: (i, 0)),
            pl.BlockSpec((BR, C), lambda i: (i, 0)),
            pl.BlockSpec(memory_space=pltpu.SMEM),
            pl.BlockSpec(memory_space=pltpu.SMEM),
        ],
        out_specs=pl.BlockSpec((BR, C), lambda i: (i, 0)),
        out_shape=jax.ShapeDtypeStruct((NPAD, C), jnp.float32),
    )(accp, packed, logits, conf_coef.reshape(1), bias_p)


def kernel(x, edge_index, dist_to_train, W_model, b_model, W_temp, conf_coef, train_a, dist1_a, bias_p):
    src = edge_index[0].astype(jnp.int32)
    dst = edge_index[1].astype(jnp.int32)
    E1 = src.shape[0]
    epp = ((E1 + 2 * NW * K - 1) // (2 * NW * K)) * (2 * NW * K)
    nchunks = epp // (16 * K)
    nchunks32 = epp // (NW * K)
    padn = epp - E1
    srcp = jnp.concatenate([src, jnp.zeros((padn,), jnp.int32)])
    dstp = jnp.concatenate([dst, jnp.full((padn,), N, jnp.int32)])

    xsplit = jnp.stack([x[:, :C // 2], x[:, C // 2:]])
    aggp, degp = _backbone_sc(xsplit, srcp, dstp, nchunks)

    dist_pad = jnp.full((NPAD, 1), 2, jnp.int32).at[:N, 0].set(dist_to_train.astype(jnp.int32))
    logits, af_pad, packed = _node_tc(aggp, degp, W_model, W_temp, b_model,
                                      dist_pad, train_a, dist1_a)

    ae, maxp = _attn_sc(af_pad, srcp, dstp, nchunks32, epp)
    accp = _soft_sc(packed, srcp, dstp, ae, maxp, nchunks32)
    res = _final_tc(accp, packed, logits, conf_coef, bias_p)
    return res[:N]


# speedup vs baseline: 1.0476x; 1.0476x over previous
import functools

import jax
import jax.numpy as jnp
from jax import lax
from jax.experimental import pallas as pl
from jax.experimental.pallas import tpu as pltpu
from jax.experimental.pallas import tpu_sc as plsc

N = 10000
C = 128
H = 8
NEG_SLOPE = 0.2

NPAD = 10240
ROWS_PER_TILE = NPAD // 16
K = 128
NW = 32

_MESH = plsc.VectorSubcoreMesh(core_axis_name="c", subcore_axis_name="s")


def _backbone_body(nchunks, xs_h, src_h, dst_h, z2d_h, z16_h, eye_h,
                   outagg_h, outdeg_h,
                   idxS0, idxD0, bufX0, idxS1, idxD1, bufX1, ones_v, sem,
                   agg_sh, deg_sh):
    c = lax.axis_index("c")
    s = lax.axis_index("s")
    row0 = s * ROWS_PER_TILE
    for i in range(8):
        pltpu.sync_copy(z2d_h, agg_sh.at[pl.ds(row0 + i * (ROWS_PER_TILE // 8), ROWS_PER_TILE // 8), :])
    pltpu.sync_copy(z16_h, deg_sh.at[pl.ds(row0, ROWS_PER_TILE), :])
    pltpu.sync_copy(eye_h.at[c], ones_v)
    plsc.subcore_barrier()

    base_w = s * (nchunks * K)

    def issue(g, idxS, idxD, bufX):
        base = base_w + g * K
        pltpu.sync_copy(src_h.at[pl.ds(base, K)], idxS)
        pltpu.sync_copy(dst_h.at[pl.ds(base, K)], idxD)
        pltpu.async_copy(xs_h.at[c].at[idxS], bufX, sem)

    def process(idxS, idxD, bufX):
        pltpu.make_async_copy(xs_h.at[c].at[idxS], bufX, sem).wait()
        pltpu.sync_copy(bufX, agg_sh.at[idxD], add=True)

        @pl.when(c == 0)
        def _():
            pltpu.sync_copy(ones_v, deg_sh.at[idxD], add=True)

        @pl.when(c == 1)
        def _():
            pltpu.sync_copy(ones_v, deg_sh.at[idxS], add=True)

    issue(0, idxS0, idxD0, bufX0)

    def pair(h, carry):
        g0 = 2 * h
        issue(g0 + 1, idxS1, idxD1, bufX1)
        process(idxS0, idxD0, bufX0)

        @pl.when(g0 + 2 < nchunks)
        def _():
            issue(g0 + 2, idxS0, idxD0, bufX0)

        process(idxS1, idxD1, bufX1)
        return carry

    lax.fori_loop(0, nchunks // 2, pair, 0)
    plsc.subcore_barrier()
    pltpu.sync_copy(agg_sh.at[pl.ds(row0, ROWS_PER_TILE), :],
                    outagg_h.at[c, pl.ds(row0, ROWS_PER_TILE), :])
    pltpu.sync_copy(deg_sh.at[pl.ds(row0, ROWS_PER_TILE), :],
                    outdeg_h.at[c, pl.ds(row0, ROWS_PER_TILE), :])


def _backbone_sc(xsplit, srcp, dstp, nchunks):
    kfn = pl.kernel(
        functools.partial(_backbone_body, nchunks),
        out_type=[
            jax.ShapeDtypeStruct((2, NPAD, C // 2), jnp.float32),
            jax.ShapeDtypeStruct((2, NPAD, 16), jnp.float32),
        ],
        mesh=_MESH,
        compiler_params=pltpu.CompilerParams(use_tc_tiling_on_sc=False),
        scratch_types=[
            pltpu.VMEM((K,), jnp.int32),
            pltpu.VMEM((K,), jnp.int32),
            pltpu.VMEM((K, C // 2), jnp.float32),
            pltpu.VMEM((K,), jnp.int32),
            pltpu.VMEM((K,), jnp.int32),
            pltpu.VMEM((K, C // 2), jnp.float32),
            pltpu.VMEM((K, 16), jnp.float32),
            pltpu.SemaphoreType.DMA,
            pltpu.VMEM_SHARED((NPAD, C // 2), jnp.float32),
            pltpu.VMEM_SHARED((NPAD, 16), jnp.float32),
        ],
    )
    z2d = jnp.zeros((ROWS_PER_TILE // 8, C // 2), jnp.float32)
    z16 = jnp.zeros((ROWS_PER_TILE, 16), jnp.float32)
    eye = jnp.zeros((2, K, 16), jnp.float32).at[0, :, 0].set(1.0).at[1, :, 1].set(1.0)
    return kfn(xsplit, srcp, dstp, z2d, z16, eye)


RG = NPAD // 16
RT = RG // 16

_GDN = lax.GatherDimensionNumbers(offset_dims=(), collapsed_slice_dims=(0,),
                                  start_index_map=(0,))


def _take16(v, perm):
    return lax.gather(v, perm[:, None], _GDN, slice_sizes=(1,),
                      mode=lax.GatherScatterMode.PROMISE_IN_BOUNDS)


def _attn_body(nchunks, af_h, src_h, dst_h, ae_h, outmax_h,
               idxS0, idxD0, bufS0, bufD0, idxS1, idxD1, bufS1, bufD1,
               aebuf, sem, locmax, red, tmpv, spmax_sh):
    c = lax.axis_index("c")
    s = lax.axis_index("s")
    w = s * 2 + c

    neg = jnp.full((16,), -3.0e38, jnp.float32)

    def initrow(i, car):
        locmax[pl.ds(i * 16, 16)] = neg
        return car

    lax.fori_loop(0, RG, initrow, 0)

    base_w = w * nchunks * K
    lane = lax.iota(jnp.int32, 16)

    def issue(g, idxS, idxD, bufS, bufD):
        base = base_w + g * K
        pltpu.sync_copy(src_h.at[pl.ds(base, K)], idxS)
        pltpu.sync_copy(dst_h.at[pl.ds(base, K)], idxD)
        pltpu.async_copy(af_h.at[idxS], bufS, sem)
        pltpu.async_copy(af_h.at[idxD], bufD, sem)

    def compute(g, idxS, idxD, bufS, bufD):
        base = base_w + g * K
        pltpu.make_async_copy(af_h.at[idxS], bufS, sem).wait()
        pltpu.make_async_copy(af_h.at[idxD], bufD, sem).wait()
        bfly = [lax.bitwise_xor(lane, sh) for sh in (1, 2, 4, 8)]
        rots = [lax.bitwise_and(lane + r, 15) for r in range(1, 16)]

        def group(g2, car2):
            e0 = g2 * 16
            vals = jnp.zeros((16,), jnp.float32)
            for j in range(16):
                i = e0 + j
                p = bufS[i, pl.ds(0, 16)] * bufD[i, pl.ds(0, 16)]
                for b in range(1, 8):
                    p = p + bufS[i, pl.ds(16 * b, 16)] * bufD[i, pl.ds(16 * b, 16)]
                for pm in bfly:
                    p = p + _take16(p, pm)
                vals = jnp.where(lane == j, p, vals)
            vals = jnp.maximum(vals, NEG_SLOPE * vals)
            aebuf[pl.ds(e0, 16)] = vals
            dvec = idxD[pl.ds(e0, 16)]
            mv = vals
            for pm in rots:
                mv = jnp.where(_take16(dvec, pm) == dvec,
                               jnp.maximum(mv, _take16(mv, pm)), mv)
            cur = plsc.load_gather(locmax, [dvec])
            plsc.store_scatter(locmax, [dvec], jnp.maximum(cur, mv))
            return car2

        lax.fori_loop(0, K // 16, group, 0)
        pltpu.sync_copy(aebuf, ae_h.at[pl.ds(base, K)])

    issue(0, idxS0, idxD0, bufS0, bufD0)

    def pair(h, car):
        g0 = 2 * h
        issue(g0 + 1, idxS1, idxD1, bufS1, bufD1)
        compute(g0, idxS0, idxD0, bufS0, bufD0)

        @pl.when(g0 + 2 < nchunks)
        def _():
            issue(g0 + 2, idxS0, idxD0, bufS0, bufD0)

        compute(g0 + 1, idxS1, idxD1, bufS1, bufD1)
        return car

    lax.fori_loop(0, nchunks // 2, pair, 0)
    pltpu.sync_copy(locmax, spmax_sh.at[s])
    plsc.subcore_barrier()
    r0 = s * RT
    pltpu.sync_copy(spmax_sh.at[0, pl.ds(r0 * 16, RT * 16)], red)
    for t in range(1, 16):
        pltpu.sync_copy(spmax_sh.at[t, pl.ds(r0 * 16, RT * 16)], tmpv)

        def mrow(i, car):
            red[pl.ds(i * 16, 16)] = jnp.maximum(red[pl.ds(i * 16, 16)], tmpv[pl.ds(i * 16, 16)])
            return car

        lax.fori_loop(0, RT, mrow, 0)
    pltpu.sync_copy(red, outmax_h.at[c, pl.ds(r0 * 16, RT * 16)])


def _attn_sc(af, srcp, dstp, nchunks, epp):
    kfn = pl.kernel(
        functools.partial(_attn_body, nchunks),
        out_type=[
            jax.ShapeDtypeStruct((epp,), jnp.float32),
            jax.ShapeDtypeStruct((2, NPAD), jnp.float32),
        ],
        mesh=_MESH,
        compiler_params=pltpu.CompilerParams(use_tc_tiling_on_sc=False,
                                             needs_layout_passes=False),
        scratch_types=[
            pltpu.VMEM((K,), jnp.int32),
            pltpu.VMEM((K,), jnp.int32),
            pltpu.VMEM((K, C), jnp.float32),
            pltpu.VMEM((K, C), jnp.float32),
            pltpu.VMEM((K,), jnp.int32),
            pltpu.VMEM((K,), jnp.int32),
            pltpu.VMEM((K, C), jnp.float32),
            pltpu.VMEM((K, C), jnp.float32),
            pltpu.VMEM((K,), jnp.float32),
            pltpu.SemaphoreType.DMA,
            pltpu.VMEM((NPAD,), jnp.float32),
            pltpu.VMEM((RT * 16,), jnp.float32),
            pltpu.VMEM((RT * 16,), jnp.float32),
            pltpu.VMEM_SHARED((16, NPAD), jnp.float32),
        ],
    )
    return kfn(af, srcp, dstp)


def _soft_body(nchunks, packed_h, src_h, dst_h, ae_h, maxp_h, z16_h, msk_h,
               outacc_h,
               idxS0, idxD0, bufP0, aev0, idxS1, idxD1, bufP1, aev1,
               sem, aml, tmpv, mskv, acc_sh):
    c = lax.axis_index("c")
    s = lax.axis_index("s")
    w = s * 2 + c
    pltpu.sync_copy(maxp_h.at[0], aml)
    pltpu.sync_copy(maxp_h.at[1], tmpv)

    def mrow(i, car):
        aml[pl.ds(i * 16, 16)] = jnp.maximum(aml[pl.ds(i * 16, 16)], tmpv[pl.ds(i * 16, 16)])
        return car

    lax.fori_loop(0, RG, mrow, 0)
    row0 = s * ROWS_PER_TILE
    pltpu.sync_copy(z16_h, acc_sh.at[pl.ds(row0, ROWS_PER_TILE), :])
    pltpu.sync_copy(msk_h, mskv)
    plsc.subcore_barrier()
    mA = mskv[0]
    mB = mskv[1]
    base_w = w * nchunks * K

    def issue(g, idxS, idxD, bufP, aev):
        base = base_w + g * K
        pltpu.sync_copy(src_h.at[pl.ds(base, K)], idxS)
        pltpu.sync_copy(dst_h.at[pl.ds(base, K)], idxD)
        pltpu.async_copy(packed_h.at[idxS], bufP, sem)
        pltpu.sync_copy(ae_h.at[pl.ds(base, K)], aev)

    def compute(idxS, idxD, bufP, aev):
        pltpu.make_async_copy(packed_h.at[idxS], bufP, sem).wait()

        def group(g2, car2):
            e0 = g2 * 16
            dvec = idxD[pl.ds(e0, 16)]
            am = plsc.load_gather(aml, [dvec])
            ex = jnp.exp(aev[pl.ds(e0, 16)] - am)
            for j in range(16):
                i = e0 + j
                t = ex[j] * mA + mB
                bufP[i, :] = bufP[i, :] * t
            return car2

        lax.fori_loop(0, K // 16, group, 0)
        pltpu.sync_copy(bufP, acc_sh.at[idxD], add=True)

    issue(0, idxS0, idxD0, bufP0, aev0)

    def pair(h, car):
        g0 = 2 * h
        issue(g0 + 1, idxS1, idxD1, bufP1, aev1)
        compute(idxS0, idxD0, bufP0, aev0)

        @pl.when(g0 + 2 < nchunks)
        def _():
            issue(g0 + 2, idxS0, idxD0, bufP0, aev0)

        compute(idxS1, idxD1, bufP1, aev1)
        return car

    lax.fori_loop(0, nchunks // 2, pair, 0)
    plsc.subcore_barrier()
    pltpu.sync_copy(acc_sh.at[pl.ds(row0, ROWS_PER_TILE), :],
                    outacc_h.at[c, pl.ds(row0, ROWS_PER_TILE), :])


def _soft_sc(packed, srcp, dstp, ae, maxp, nchunks):
    kfn = pl.kernel(
        functools.partial(_soft_body, nchunks),
        out_type=jax.ShapeDtypeStruct((2, NPAD, 16), jnp.float32),
        mesh=_MESH,
        compiler_params=pltpu.CompilerParams(use_tc_tiling_on_sc=False,
                                             needs_layout_passes=False),
        scratch_types=[
            pltpu.VMEM((K,), jnp.int32),
            pltpu.VMEM((K,), jnp.int32),
            pltpu.VMEM((K, 16), jnp.float32),
            pltpu.VMEM((K,), jnp.float32),
            pltpu.VMEM((K,), jnp.int32),
            pltpu.VMEM((K,), jnp.int32),
            pltpu.VMEM((K, 16), jnp.float32),
            pltpu.VMEM((K,), jnp.float32),
            pltpu.SemaphoreType.DMA,
            pltpu.VMEM((NPAD,), jnp.float32),
            pltpu.VMEM((NPAD,), jnp.float32),
            pltpu.VMEM((2, 16), jnp.float32),
            pltpu.VMEM_SHARED((NPAD, 16), jnp.float32),
        ],
    )
    z16 = jnp.zeros((ROWS_PER_TILE, 16), jnp.float32)
    msk = jnp.zeros((2, 16), jnp.float32).at[0, 0:8].set(1.0).at[0, 9].set(1.0).at[1, 8].set(1.0)
    return kfn(packed, srcp, dstp, ae, maxp, z16, msk)


BR = 512


def _sort_lanes(v):
    li = lax.broadcasted_iota(jnp.int32, v.shape, 1)
    k = 2
    while k <= 128:
        j = k // 2
        while j >= 1:
            bitj0 = (li & j) == 0
            p = jnp.where(bitj0, jnp.roll(v, -j, axis=1), jnp.roll(v, j, axis=1))
            dirup = (li & k) == 0
            keepmin = bitj0 == dirup
            v = jnp.where(keepmin, jnp.minimum(v, p), jnp.maximum(v, p))
            j //= 2
        k *= 2
    return v


def _node_body(aggp_ref, degp_ref, w_ref, wt_ref, b_ref, dist_ref, ta_ref, da_ref,
               logits_ref, af_ref, packed_ref):
    agg = jnp.concatenate([aggp_ref[0], aggp_ref[1]], axis=1)
    din = degp_ref[0, :, 0:1]
    dout = degp_ref[1, :, 1:2]
    logits = (agg / jnp.maximum(din, 1.0)) @ w_ref[...] + b_ref[...]
    logits_ref[...] = logits
    mn = jnp.min(logits, axis=1, keepdims=True)
    mx = jnp.max(logits, axis=1, keepdims=True)
    normalized = (logits - mn) / jnp.maximum(mx - mn, 1e-30)
    x_sorted = _sort_lanes(normalized)
    temp = jax.lax.dot(x_sorted, wt_ref[...], preferred_element_type=jnp.float32)
    m = jnp.max(logits, axis=1, keepdims=True)
    conf = 1.0 / jnp.sum(jnp.exp(logits - m), axis=1, keepdims=True)
    dist = dist_ref[...]
    a = jnp.where(dist == 0, ta_ref[0], jnp.where(dist == 1, da_ref[0], 1.0))
    af_ref[...] = logits * (1.0 / a)
    dinv = jnp.where(dout > 0, 1.0 / dout, 0.0)
    zs = jnp.zeros_like(temp[:, 0:4])
    packed_ref[...] = jnp.concatenate(
        [temp[:, 0:8] * a, conf, jnp.ones_like(conf), din, dinv, zs], axis=1)


def _node_tc(aggp, degp, W_model, W_temp, b_model, dist_pad, train_a, dist1_a):
    wt_pad = jnp.zeros((C, C), jnp.float32).at[:, :H].set(W_temp)
    return pl.pallas_call(
        _node_body,
        grid=(NPAD // BR,),
        in_specs=[
            pl.BlockSpec((2, BR, C // 2), lambda i: (0, i, 0)),
            pl.BlockSpec((2, BR, 16), lambda i: (0, i, 0)),
            pl.BlockSpec((C, C), lambda i: (0, 0)),
            pl.BlockSpec((C, C), lambda i: (0, 0)),
            pl.BlockSpec((1, C), lambda i: (0, 0)),
            pl.BlockSpec((BR, 1), lambda i: (i, 0)),
            pl.BlockSpec(memory_space=pltpu.SMEM),
            pl.BlockSpec(memory_space=pltpu.SMEM),
        ],
        out_specs=[
            pl.BlockSpec((BR, C), lambda i: (i, 0)),
            pl.BlockSpec((BR, C), lambda i: (i, 0)),
            pl.BlockSpec((BR, 16), lambda i: (i, 0)),
        ],
        out_shape=[
            jax.ShapeDtypeStruct((NPAD, C), jnp.float32),
            jax.ShapeDtypeStruct((NPAD, C), jnp.float32),
            jax.ShapeDtypeStruct((NPAD, 16), jnp.float32),
        ],
    )(aggp, degp, W_model, wt_pad, b_model.reshape(1, C), dist_pad,
      train_a, dist1_a)


def _final_body(accp_ref, packed_ref, logits_ref, coef_ref, bias_ref, out_ref):
    accs = accp_ref[0] + accp_ref[1]
    sim = accs[:, 0:8] / accs[:, 9:10]
    conf = packed_ref[:, 8:9]
    din = packed_ref[:, 10:11]
    dinv = packed_ref[:, 11:12]
    dconf = din * conf - accs[:, 8:9]
    out = jax.nn.softplus(sim + coef_ref[0] * dconf * dinv)
    t = jnp.sum(out, axis=1, keepdims=True) * (1.0 / H) + bias_ref[0]
    out_ref[...] = logits_ref[...] / t


def _final_tc(accp, packed, logits, conf_coef, bias_p):
    return pl.pallas_call(
        _final_body,
        grid=(NPAD // BR,),
        in_specs=[
            pl.BlockSpec((2, BR, 16), lambda i: (0, i, 0)),
            pl.BlockSpec((BR, 16), lambda i: (i, 0)),
            pl.BlockSpec((BR, C), lambda i: (i, 0)),
            pl.BlockSpec(memory_space=pltpu.SMEM),
            pl.BlockSpec(memory_space=pltpu.SMEM),
        ],
        out_specs=pl.BlockSpec((BR, C), lambda i: (i, 0)),
        out_shape=jax.ShapeDtypeStruct((NPAD, C), jnp.float32),
    )(accp, packed, logits, conf_coef.reshape(1), bias_p)


def kernel(x, edge_index, dist_to_train, W_model, b_model, W_temp, conf_coef, train_a, dist1_a, bias_p):
    src = edge_index[0].astype(jnp.int32)
    dst = edge_index[1].astype(jnp.int32)
    E1 = src.shape[0]
    epp = ((E1 + 2 * NW * K - 1) // (2 * NW * K)) * (2 * NW * K)
    nchunks = epp // (16 * K)
    nchunks32 = epp // (NW * K)
    padn = epp - E1
    srcp = jnp.concatenate([src, jnp.zeros((padn,), jnp.int32)])
    dstp = jnp.concatenate([dst, jnp.full((padn,), N, jnp.int32)])

    xsplit = jnp.stack([x[:, :C // 2], x[:, C // 2:]])
    aggp, degp = _backbone_sc(xsplit, srcp, dstp, nchunks)

    dist_pad = jnp.full((NPAD, 1), 2, jnp.int32).at[:N, 0].set(dist_to_train.astype(jnp.int32))
    logits, af_pad, packed = _node_tc(aggp, degp, W_model, W_temp, b_model,
                                      dist_pad, train_a, dist1_a)

    ae, maxp = _attn_sc(af_pad, srcp, dstp, nchunks32, epp)
    accp = _soft_sc(packed, srcp, dstp, ae, maxp, nchunks32)
    res = _final_tc(accp, packed, logits, conf_coef, bias_p)
    return res[:N]
